# baseline (speedup 1.0000x reference)
import functools

import jax
import jax.numpy as jnp
from jax import lax
from jax.experimental import pallas as pl
from jax.experimental.pallas import tpu as pltpu
from jax.experimental.pallas import tpu_sc as plsc

N_ITEMS = 10000
N_PERSONS = 100000
N_GRADES = 5
N_RESP = 1048576

NC, NS, L = 2, 16, 16
NW = NC * NS
CH = 2048
_HALF = N_PERSONS // 2

_LOG2PI = 1.8378770664093453


def _softplus(x):
    return jnp.maximum(x, 0.0) + jnp.log1p(jnp.exp(-jnp.abs(x)))


def _bf16_bits(x):
    return lax.bitcast_convert_type(
        x.astype(jnp.bfloat16), jnp.uint16).astype(jnp.int32)


def _prep_body(a_ref, bb_ref, d0_ref, d1_ref, d2_ref, t_ref,
               a_out, b0_out, b1_out, b2_out, b3_out, t2_out, prior_out):
    a = _softplus(a_ref[...])
    b0 = bb_ref[...]
    b1 = b0 + _softplus(d0_ref[...])
    b2 = b1 + _softplus(d1_ref[...])
    b3 = b2 + _softplus(d2_ref[...])
    a_out[...] = a
    b0_out[...] = b0
    b1_out[...] = b1
    b2_out[...] = b2
    b3_out[...] = b3
    t = t_ref[...]
    t2_out[...] = _bf16_bits(t[:_HALF]) | (_bf16_bits(t[_HALF:]) << 16)
    n_elem = N_ITEMS + N_ITEMS * (N_GRADES - 1) + N_PERSONS
    sq = (jnp.sum(a * a) + jnp.sum(b0 * b0) + jnp.sum(b1 * b1)
          + jnp.sum(b2 * b2) + jnp.sum(b3 * b3) + jnp.sum(t * t))
    prior_out[0, 0] = -0.5 * _LOG2PI * n_elem - 0.5 * sq


def _prep(a_, b_base_, b_diff_, t):
    vec = jax.ShapeDtypeStruct((N_ITEMS,), jnp.float32)
    return pl.pallas_call(
        _prep_body,
        out_shape=(vec, vec, vec, vec, vec,
                   jax.ShapeDtypeStruct((_HALF,), jnp.int32),
                   jax.ShapeDtypeStruct((1, 1), jnp.float32)),
        out_specs=(pl.BlockSpec(), pl.BlockSpec(), pl.BlockSpec(),
                   pl.BlockSpec(), pl.BlockSpec(), pl.BlockSpec(),
                   pl.BlockSpec(memory_space=pltpu.SMEM)),
    )(a_, b_base_.reshape(N_ITEMS),
      b_diff_[:, 0], b_diff_[:, 1], b_diff_[:, 2], t)


def _make_sc_body(n_resp):
    per_w = n_resp // NW
    n_ch = per_w // CH

    def _sc_body(ir_h, pe_h, a_h, b0_h, b1_h, b2_h, b3_h, t2_h, p_h,
                 a_v, b_v, t2_v, ir0, ir1, pe0, pe1, pb0, pb1,
                 si0, si1, sp0, sp1):
        wid = lax.axis_index("s") * NC + lax.axis_index("c")
        pltpu.sync_copy(a_h, a_v)
        for k, bk in enumerate((b0_h, b1_h, b2_h, b3_h)):
            pltpu.sync_copy(bk, b_v.at[pl.ds(k * N_ITEMS, N_ITEMS)])
        pltpu.sync_copy(t2_h, t2_v)
        base = wid * per_w

        ibufs = ((ir0, pe0), (ir1, pe1))
        pbufs = (pb0, pb1)
        isems = (si0, si1)
        psems = (sp0, sp1)

        def fire_idx(ci, b):
            off = base + lax.rem(ci, n_ch) * CH
            for src, dst in zip((ir_h, pe_h), ibufs[b]):
                pltpu.async_copy(src.at[pl.ds(off, CH)], dst, isems[b])

        fire_idx(jnp.int32(0), 0)
        fire_idx(jnp.int32(1), 1)

        def pair(k, carry):
            for b in range(2):
                ci = 2 * k + b
                (ir_v, pe_v), pb = ibufs[b], pbufs[b]
                for src, dst in zip((ir_h, pe_h), ibufs[b]):
                    pltpu.make_async_copy(
                        src.at[pl.ds(0, CH)], dst, isems[b]).wait()
                @pl.when(k >= 1)
                def _():
                    pltpu.make_async_copy(
                        pb, p_h.at[pl.ds(0, CH)], psems[b]).wait()

                def prob16(sl):
                    ir = ir_v[sl]
                    pe = pe_v[sl]
                    it = ir >> 3
                    rs = ir & 7
                    hi = pe >= _HALF
                    tw = plsc.load_gather(
                        t2_v, [pe - jnp.where(hi, _HALF, 0)])
                    bits = jnp.where(hi, tw & jnp.int32(-65536), tw << 16)
                    tv = plsc.bitcast(bits, jnp.float32)
                    av = plsc.load_gather(a_v, [it])
                    bi = it + rs * N_ITEMS
                    bu = plsc.load_gather(
                        b_v, [jnp.maximum(bi - 2 * N_ITEMS, 0)])
                    bl = plsc.load_gather(
                        b_v, [jnp.minimum(bi - N_ITEMS, N_ITEMS * 4 - 1)])
                    x = av * tv
                    eu = jnp.exp(jnp.minimum(av * bu - x, 40.0))
                    el = jnp.exp(jnp.minimum(av * bl - x, 40.0))
                    eu = jnp.where(rs == 1, 0.0, eu)
                    el = jnp.where(rs == N_GRADES, 1e30, el)
                    p = (el - eu) / ((1.0 + eu) * (1.0 + el))
                    return jnp.clip(p, 1e-12, 1.0)

                @plsc.parallel_loop(0, CH // (2 * L), unroll=4)
                def step(s):
                    p0 = prob16(pl.ds(s * (2 * L), L))
                    p1 = prob16(pl.ds(s * (2 * L) + L, L))
                    pb[pl.ds(s * (2 * L), 2 * L)] = plsc.pack(
                        p0, p1, format=plsc.PackFormat.INTERLEAVED)

                pltpu.async_copy(
                    pb, p_h.at[pl.ds(base + ci * CH, CH)], psems[b])
                fire_idx(ci + 2, b)
            return carry

        lax.fori_loop(0, n_ch // 2, pair, 0)

        for b in range(2):
            pltpu.make_async_copy(
                pbufs[b], p_h.at[pl.ds(0, CH)], psems[b]).wait()
            for src, dst in zip((ir_h, pe_h), ibufs[b]):
                pltpu.make_async_copy(
                    src.at[pl.ds(0, CH)], dst, isems[b]).wait()

    return _sc_body


@functools.cache
def _make_sc_gather(n_resp):
    mesh = plsc.VectorSubcoreMesh(
        core_axis_name="c", subcore_axis_name="s",
        num_cores=NC, num_subcores=NS)
    return functools.partial(
        pl.kernel,
        out_type=jax.ShapeDtypeStruct((n_resp,), jnp.bfloat16),
        mesh=mesh,
        scratch_types=[
            pltpu.VMEM((N_ITEMS,), jnp.float32),
            pltpu.VMEM((N_ITEMS * 4,), jnp.float32),
            pltpu.VMEM((N_PERSONS // 2,), jnp.int32),
            pltpu.VMEM((CH,), jnp.int32),
            pltpu.VMEM((CH,), jnp.int32),
            pltpu.VMEM((CH,), jnp.int32),
            pltpu.VMEM((CH,), jnp.int32),
            pltpu.VMEM((CH,), jnp.bfloat16),
            pltpu.VMEM((CH,), jnp.bfloat16),
            pltpu.SemaphoreType.DMA,
            pltpu.SemaphoreType.DMA,
            pltpu.SemaphoreType.DMA,
            pltpu.SemaphoreType.DMA,
        ],
        compiler_params=pltpu.CompilerParams(needs_layout_passes=False),
    )(_make_sc_body(n_resp))


_RG = 8


def _reduce_body(p_ref, prior_ref, out_ref, acc_ref):
    i = pl.program_id(0)

    @pl.when(i == 0)
    def _():
        acc_ref[0] = 0.0

    acc_ref[0] += jnp.sum(jnp.log(p_ref[...].astype(jnp.float32)))

    @pl.when(i == _RG - 1)
    def _():
        out_ref[0, 0] = acc_ref[0] + prior_ref[0, 0]


def _reduce(p2d, prior):
    rows = p2d.shape[0] // _RG
    return pl.pallas_call(
        _reduce_body,
        grid=(_RG,),
        out_shape=jax.ShapeDtypeStruct((1, 1), jnp.float32),
        in_specs=(
            pl.BlockSpec((rows, 128), lambda i: (i, 0)),
            pl.BlockSpec(memory_space=pltpu.SMEM),
        ),
        out_specs=pl.BlockSpec(memory_space=pltpu.SMEM),
        scratch_shapes=[pltpu.SMEM((1,), jnp.float32)],
    )(p2d, prior)


def kernel(indices, a_, b_base_, b_diff_, t):
    h = N_RESP // 2
    ir0 = (indices[:h, 0] << 3) | indices[:h, 2]
    pe0 = indices[:h, 1]
    ir1 = (indices[h:, 0] << 3) | indices[h:, 2]
    pe1 = indices[h:, 1]
    a, b0, b1, b2, b3, t2, prior = _prep(a_, b_base_, b_diff_, t)
    sc = _make_sc_gather(h)
    p0 = sc(ir0, pe0, a, b0, b1, b2, b3, t2)
    p1 = sc(ir1, pe1, a, b0, b1, b2, b3, t2)
    zero = jnp.zeros((1, 1), jnp.float32)
    ll0 = _reduce(p0.reshape(h // 128, 128), zero)
    out = _reduce(p1.reshape(h // 128, 128), prior + ll0)
    return -out.reshape(())


# speedup vs baseline: 1.0351x; 1.0351x over previous
import functools

import jax
import jax.numpy as jnp
from jax import lax
from jax.experimental import pallas as pl
from jax.experimental.pallas import tpu as pltpu
from jax.experimental.pallas import tpu_sc as plsc

N_ITEMS = 10000
N_PERSONS = 100000
N_GRADES = 5
N_RESP = 1048576

NC, NS, L = 2, 16, 16
NW = NC * NS
CH = 2048
_HALF = N_PERSONS // 2

_LOG2PI = 1.8378770664093453


def _softplus(x):
    return jnp.maximum(x, 0.0) + jnp.log1p(jnp.exp(-jnp.abs(x)))


def _bf16_bits(x):
    return lax.bitcast_convert_type(
        x.astype(jnp.bfloat16), jnp.uint16).astype(jnp.int32)


def _prep_body(a_ref, bb_ref, d0_ref, d1_ref, d2_ref, t_ref,
               a_out, b0_out, b1_out, b2_out, b3_out, t2_out, prior_out):
    a = _softplus(a_ref[...])
    b0 = bb_ref[...]
    b1 = b0 + _softplus(d0_ref[...])
    b2 = b1 + _softplus(d1_ref[...])
    b3 = b2 + _softplus(d2_ref[...])
    a_out[...] = a
    b0_out[...] = b0
    b1_out[...] = b1
    b2_out[...] = b2
    b3_out[...] = b3
    t = t_ref[...]
    t2_out[...] = _bf16_bits(t[:_HALF]) | (_bf16_bits(t[_HALF:]) << 16)
    n_elem = N_ITEMS + N_ITEMS * (N_GRADES - 1) + N_PERSONS
    sq = (jnp.sum(a * a) + jnp.sum(b0 * b0) + jnp.sum(b1 * b1)
          + jnp.sum(b2 * b2) + jnp.sum(b3 * b3) + jnp.sum(t * t))
    prior_out[0, 0] = -0.5 * _LOG2PI * n_elem - 0.5 * sq


def _prep(a_, b_base_, b_diff_, t):
    vec = jax.ShapeDtypeStruct((N_ITEMS,), jnp.float32)
    return pl.pallas_call(
        _prep_body,
        out_shape=(vec, vec, vec, vec, vec,
                   jax.ShapeDtypeStruct((_HALF,), jnp.int32),
                   jax.ShapeDtypeStruct((1, 1), jnp.float32)),
        out_specs=(pl.BlockSpec(), pl.BlockSpec(), pl.BlockSpec(),
                   pl.BlockSpec(), pl.BlockSpec(), pl.BlockSpec(),
                   pl.BlockSpec(memory_space=pltpu.SMEM)),
    )(a_, b_base_.reshape(N_ITEMS),
      b_diff_[:, 0], b_diff_[:, 1], b_diff_[:, 2], t)


def _make_sc_body(n_resp, with_dep):
    per_w = n_resp // NW
    n_ch = per_w // CH

    def _sc_body(*refs):
        if with_dep:
            refs = refs[1:]
        (ir_h, pe_h, a_h, b0_h, b1_h, b2_h, b3_h, t2_h, p_h,
         a_v, b_v, t2_v, ir0, ir1, pe0, pe1, pb0, pb1,
         si0, si1, sp0, sp1) = refs
        wid = lax.axis_index("s") * NC + lax.axis_index("c")
        pltpu.sync_copy(a_h, a_v)
        for k, bk in enumerate((b0_h, b1_h, b2_h, b3_h)):
            pltpu.sync_copy(bk, b_v.at[pl.ds(k * N_ITEMS, N_ITEMS)])
        pltpu.sync_copy(t2_h, t2_v)
        base = wid * per_w

        ibufs = ((ir0, pe0), (ir1, pe1))
        pbufs = (pb0, pb1)
        isems = (si0, si1)
        psems = (sp0, sp1)

        def fire_idx(ci, b):
            off = base + lax.rem(ci, n_ch) * CH
            for src, dst in zip((ir_h, pe_h), ibufs[b]):
                pltpu.async_copy(src.at[pl.ds(off, CH)], dst, isems[b])

        fire_idx(jnp.int32(0), 0)
        fire_idx(jnp.int32(1), 1)

        def pair(k, carry):
            for b in range(2):
                ci = 2 * k + b
                (ir_v, pe_v), pb = ibufs[b], pbufs[b]
                for src, dst in zip((ir_h, pe_h), ibufs[b]):
                    pltpu.make_async_copy(
                        src.at[pl.ds(0, CH)], dst, isems[b]).wait()
                @pl.when(k >= 1)
                def _():
                    pltpu.make_async_copy(
                        pb, p_h.at[pl.ds(0, CH)], psems[b]).wait()

                def prob16(sl):
                    ir = ir_v[sl]
                    pe = pe_v[sl]
                    it = ir >> 3
                    rs = ir & 7
                    hi = pe >= _HALF
                    tw = plsc.load_gather(
                        t2_v, [pe - jnp.where(hi, _HALF, 0)])
                    bits = jnp.where(hi, tw & jnp.int32(-65536), tw << 16)
                    tv = plsc.bitcast(bits, jnp.float32)
                    av = plsc.load_gather(a_v, [it])
                    bi = it + rs * N_ITEMS
                    bu = plsc.load_gather(
                        b_v, [jnp.maximum(bi - 2 * N_ITEMS, 0)])
                    bl = plsc.load_gather(
                        b_v, [jnp.minimum(bi - N_ITEMS, N_ITEMS * 4 - 1)])
                    su = 1.0 / (1.0 + jnp.exp(av * (bu - tv)))
                    slo = 1.0 / (1.0 + jnp.exp(av * (bl - tv)))
                    upper = jnp.where(rs == 1, 1.0, su)
                    lower = jnp.where(rs == N_GRADES, 0.0, slo)
                    return jnp.clip(upper - lower, 1e-12, 1.0)

                @plsc.parallel_loop(0, CH // (2 * L), unroll=4)
                def step(s):
                    p0 = prob16(pl.ds(s * (2 * L), L))
                    p1 = prob16(pl.ds(s * (2 * L) + L, L))
                    pb[pl.ds(s * (2 * L), 2 * L)] = plsc.pack(
                        p0, p1, format=plsc.PackFormat.INTERLEAVED)

                pltpu.async_copy(
                    pb, p_h.at[pl.ds(base + ci * CH, CH)], psems[b])
                fire_idx(ci + 2, b)
            return carry

        lax.fori_loop(0, n_ch // 2, pair, 0)

        for b in range(2):
            pltpu.make_async_copy(
                pbufs[b], p_h.at[pl.ds(0, CH)], psems[b]).wait()
            for src, dst in zip((ir_h, pe_h), ibufs[b]):
                pltpu.make_async_copy(
                    src.at[pl.ds(0, CH)], dst, isems[b]).wait()

    return _sc_body


@functools.cache
def _make_sc_gather(n_resp, with_dep=False):
    mesh = plsc.VectorSubcoreMesh(
        core_axis_name="c", subcore_axis_name="s",
        num_cores=NC, num_subcores=NS)
    return functools.partial(
        pl.kernel,
        out_type=jax.ShapeDtypeStruct((n_resp,), jnp.bfloat16),
        mesh=mesh,
        scratch_types=[
            pltpu.VMEM((N_ITEMS,), jnp.float32),
            pltpu.VMEM((N_ITEMS * 4,), jnp.float32),
            pltpu.VMEM((N_PERSONS // 2,), jnp.int32),
            pltpu.VMEM((CH,), jnp.int32),
            pltpu.VMEM((CH,), jnp.int32),
            pltpu.VMEM((CH,), jnp.int32),
            pltpu.VMEM((CH,), jnp.int32),
            pltpu.VMEM((CH,), jnp.bfloat16),
            pltpu.VMEM((CH,), jnp.bfloat16),
            pltpu.SemaphoreType.DMA,
            pltpu.SemaphoreType.DMA,
            pltpu.SemaphoreType.DMA,
            pltpu.SemaphoreType.DMA,
        ],
        compiler_params=pltpu.CompilerParams(needs_layout_passes=False),
    )(_make_sc_body(n_resp, with_dep))


_RG = 8


def _reduce_body(p_ref, prior_ref, out_ref, acc_ref):
    i = pl.program_id(0)

    @pl.when(i == 0)
    def _():
        acc_ref[0] = 0.0

    acc_ref[0] += jnp.sum(jnp.log(p_ref[...].astype(jnp.float32)))

    @pl.when(i == _RG - 1)
    def _():
        out_ref[0, 0] = acc_ref[0] + prior_ref[0, 0]


def _reduce(p2d, prior):
    rows = p2d.shape[0] // _RG
    return pl.pallas_call(
        _reduce_body,
        grid=(_RG,),
        out_shape=jax.ShapeDtypeStruct((1, 1), jnp.float32),
        in_specs=(
            pl.BlockSpec((rows, 128), lambda i: (i, 0)),
            pl.BlockSpec(memory_space=pltpu.SMEM),
        ),
        out_specs=pl.BlockSpec(memory_space=pltpu.SMEM),
        scratch_shapes=[pltpu.SMEM((1,), jnp.float32)],
    )(p2d, prior)


def kernel(indices, a_, b_base_, b_diff_, t):
    h = N_RESP // 2
    ir0 = (indices[:h, 0] << 3) | indices[:h, 2]
    pe0 = indices[:h, 1]
    ir1 = (indices[h:, 0] << 3) | indices[h:, 2]
    pe1 = indices[h:, 1]
    a, b0, b1, b2, b3, t2, prior = _prep(a_, b_base_, b_diff_, t)
    p0 = _make_sc_gather(h)(ir0, pe0, a, b0, b1, b2, b3, t2)
    p1 = _make_sc_gather(h, with_dep=True)(
        p0, ir1, pe1, a, b0, b1, b2, b3, t2)
    zero = jnp.zeros((1, 1), jnp.float32)
    ll0 = _reduce(p0.reshape(h // 128, 128), zero)
    out = _reduce(p1.reshape(h // 128, 128), prior + ll0)
    return -out.reshape(())


# speedup vs baseline: 1.1710x; 1.1313x over previous
import functools

import jax
import jax.numpy as jnp
from jax import lax
from jax.experimental import pallas as pl
from jax.experimental.pallas import tpu as pltpu
from jax.experimental.pallas import tpu_sc as plsc

N_ITEMS = 10000
N_PERSONS = 100000
N_GRADES = 5
N_RESP = 1048576

NC, NS, L = 2, 16, 16
NW = NC * NS
CH = 2048
_HALF = N_PERSONS // 2

_LOG2PI = 1.8378770664093453


def _softplus(x):
    return jnp.maximum(x, 0.0) + jnp.log1p(jnp.exp(-jnp.abs(x)))


def _bf16_bits(x):
    return lax.bitcast_convert_type(
        x.astype(jnp.bfloat16), jnp.uint16).astype(jnp.int32)


def _prep_body(a_ref, bb_ref, d0_ref, d1_ref, d2_ref, t_ref,
               a_out, b0_out, b1_out, b2_out, b3_out, t2_out, prior_out):
    a = _softplus(a_ref[...])
    b0 = bb_ref[...]
    b1 = b0 + _softplus(d0_ref[...])
    b2 = b1 + _softplus(d1_ref[...])
    b3 = b2 + _softplus(d2_ref[...])
    a_out[...] = a
    b0_out[...] = b0
    b1_out[...] = b1
    b2_out[...] = b2
    b3_out[...] = b3
    t = t_ref[...]
    t2_out[...] = _bf16_bits(t[:_HALF]) | (_bf16_bits(t[_HALF:]) << 16)
    n_elem = N_ITEMS + N_ITEMS * (N_GRADES - 1) + N_PERSONS
    sq = (jnp.sum(a * a) + jnp.sum(b0 * b0) + jnp.sum(b1 * b1)
          + jnp.sum(b2 * b2) + jnp.sum(b3 * b3) + jnp.sum(t * t))
    prior_out[0, 0] = -0.5 * _LOG2PI * n_elem - 0.5 * sq


def _prep(a_, b_base_, b_diff_, t):
    vec = jax.ShapeDtypeStruct((N_ITEMS,), jnp.float32)
    return pl.pallas_call(
        _prep_body,
        out_shape=(vec, vec, vec, vec, vec,
                   jax.ShapeDtypeStruct((_HALF,), jnp.int32),
                   jax.ShapeDtypeStruct((1, 1), jnp.float32)),
        out_specs=(pl.BlockSpec(), pl.BlockSpec(), pl.BlockSpec(),
                   pl.BlockSpec(), pl.BlockSpec(), pl.BlockSpec(),
                   pl.BlockSpec(memory_space=pltpu.SMEM)),
    )(a_, b_base_.reshape(N_ITEMS),
      b_diff_[:, 0], b_diff_[:, 1], b_diff_[:, 2], t)


def _make_sc_body(n_resp, with_dep):
    per_w = n_resp // NW
    n_ch = per_w // CH

    def _sc_body(*refs):
        if with_dep:
            refs = refs[1:]
        (ir_h, pe_h, a_h, b0_h, b1_h, b2_h, b3_h, t2_h, p_h,
         a_v, b_v, t2_v, ir0, ir1, pe0, pe1, pb0, pb1,
         si0, si1, sp0, sp1) = refs
        wid = lax.axis_index("s") * NC + lax.axis_index("c")
        pltpu.sync_copy(a_h, a_v)
        for k, bk in enumerate((b0_h, b1_h, b2_h, b3_h)):
            pltpu.sync_copy(bk, b_v.at[pl.ds(k * N_ITEMS, N_ITEMS)])
        pltpu.sync_copy(t2_h, t2_v)
        base = wid * per_w

        ibufs = ((ir0, pe0), (ir1, pe1))
        pbufs = (pb0, pb1)
        isems = (si0, si1)
        psems = (sp0, sp1)

        def fire_idx(ci, b):
            off = base + lax.rem(ci, n_ch) * CH
            for src, dst in zip((ir_h, pe_h), ibufs[b]):
                pltpu.async_copy(src.at[pl.ds(off, CH)], dst, isems[b])

        fire_idx(jnp.int32(0), 0)
        fire_idx(jnp.int32(1), 1)

        def pair(k, carry):
            for b in range(2):
                ci = 2 * k + b
                (ir_v, pe_v), pb = ibufs[b], pbufs[b]
                for src, dst in zip((ir_h, pe_h), ibufs[b]):
                    pltpu.make_async_copy(
                        src.at[pl.ds(0, CH)], dst, isems[b]).wait()
                @pl.when(k >= 1)
                def _():
                    pltpu.make_async_copy(
                        pb, p_h.at[pl.ds(0, CH)], psems[b]).wait()

                def prob16(sl):
                    ir = ir_v[sl]
                    pe = pe_v[sl]
                    it = ir >> 3
                    rs = ir & 7
                    hi = pe >= _HALF
                    tw = plsc.load_gather(
                        t2_v, [pe - jnp.where(hi, _HALF, 0)])
                    bits = jnp.where(hi, tw & jnp.int32(-65536), tw << 16)
                    tv = plsc.bitcast(bits, jnp.float32)
                    av = plsc.load_gather(a_v, [it])
                    bi = it + rs * N_ITEMS
                    bu = plsc.load_gather(
                        b_v, [jnp.maximum(bi - 2 * N_ITEMS, 0)])
                    bl = plsc.load_gather(
                        b_v, [jnp.minimum(bi - N_ITEMS, N_ITEMS * 4 - 1)])
                    su = 1.0 / (1.0 + jnp.exp(av * (bu - tv)))
                    slo = 1.0 / (1.0 + jnp.exp(av * (bl - tv)))
                    upper = jnp.where(rs == 1, 1.0, su)
                    lower = jnp.where(rs == N_GRADES, 0.0, slo)
                    return jnp.clip(upper - lower, 1e-12, 1.0)

                @plsc.parallel_loop(0, CH // (2 * L), unroll=4)
                def step(s):
                    p0 = prob16(pl.ds(s * (2 * L), L))
                    p1 = prob16(pl.ds(s * (2 * L) + L, L))
                    pb[pl.ds(s * (2 * L), 2 * L)] = plsc.pack(
                        p0, p1, format=plsc.PackFormat.INTERLEAVED)

                pltpu.async_copy(
                    pb, p_h.at[pl.ds(base + ci * CH, CH)], psems[b])
                fire_idx(ci + 2, b)
            return carry

        lax.fori_loop(0, n_ch // 2, pair, 0)

        for b in range(2):
            pltpu.make_async_copy(
                pbufs[b], p_h.at[pl.ds(0, CH)], psems[b]).wait()
            for src, dst in zip((ir_h, pe_h), ibufs[b]):
                pltpu.make_async_copy(
                    src.at[pl.ds(0, CH)], dst, isems[b]).wait()

    return _sc_body


@functools.cache
def _make_sc_gather(n_resp, with_dep=False):
    mesh = plsc.VectorSubcoreMesh(
        core_axis_name="c", subcore_axis_name="s",
        num_cores=NC, num_subcores=NS)
    return functools.partial(
        pl.kernel,
        out_type=jax.ShapeDtypeStruct((n_resp,), jnp.bfloat16),
        mesh=mesh,
        scratch_types=[
            pltpu.VMEM((N_ITEMS,), jnp.float32),
            pltpu.VMEM((N_ITEMS * 4,), jnp.float32),
            pltpu.VMEM((N_PERSONS // 2,), jnp.int32),
            pltpu.VMEM((CH,), jnp.int32),
            pltpu.VMEM((CH,), jnp.int32),
            pltpu.VMEM((CH,), jnp.int32),
            pltpu.VMEM((CH,), jnp.int32),
            pltpu.VMEM((CH,), jnp.bfloat16),
            pltpu.VMEM((CH,), jnp.bfloat16),
            pltpu.SemaphoreType.DMA,
            pltpu.SemaphoreType.DMA,
            pltpu.SemaphoreType.DMA,
            pltpu.SemaphoreType.DMA,
        ],
        compiler_params=pltpu.CompilerParams(needs_layout_passes=False),
    )(_make_sc_body(n_resp, with_dep))


_RG = 16


def _reduce_body(p_ref, prior_ref, out_ref, acc_ref):
    i = pl.program_id(0)

    @pl.when(i == 0)
    def _():
        acc_ref[0] = 0.0

    acc_ref[0] += jnp.sum(jnp.log(p_ref[...].astype(jnp.float32)))

    @pl.when(i == _RG - 1)
    def _():
        out_ref[0, 0] = acc_ref[0] + prior_ref[0, 0]


def _reduce(p2d, prior):
    rows = p2d.shape[0] // _RG
    return pl.pallas_call(
        _reduce_body,
        grid=(_RG,),
        out_shape=jax.ShapeDtypeStruct((1, 1), jnp.float32),
        in_specs=(
            pl.BlockSpec((rows, 128), lambda i: (i, 0)),
            pl.BlockSpec(memory_space=pltpu.SMEM),
        ),
        out_specs=pl.BlockSpec(memory_space=pltpu.SMEM),
        scratch_shapes=[pltpu.SMEM((1,), jnp.float32)],
    )(p2d, prior)


def kernel(indices, a_, b_base_, b_diff_, t):
    itemresp = (indices[:, 0] << 3) | indices[:, 2]
    person = indices[:, 1]
    a, b0, b1, b2, b3, t2, prior = _prep(a_, b_base_, b_diff_, t)
    p = _make_sc_gather(N_RESP)(itemresp, person, a, b0, b1, b2, b3, t2)
    out = _reduce(p.reshape(N_RESP // 128, 128), prior)
    return -out.reshape(())


# speedup vs baseline: 1.2003x; 1.0250x over previous
import functools

import jax
import jax.numpy as jnp
from jax import lax
from jax.experimental import pallas as pl
from jax.experimental.pallas import tpu as pltpu
from jax.experimental.pallas import tpu_sc as plsc

N_ITEMS = 10000
N_PERSONS = 100000
N_GRADES = 5
N_RESP = 1048576

NC, NS, L = 2, 16, 16
NW = NC * NS
CH = 2048
_HALF = N_PERSONS // 2

_LOG2PI = 1.8378770664093453


def _softplus(x):
    return jnp.maximum(x, 0.0) + jnp.log1p(jnp.exp(-jnp.abs(x)))


def _bf16_bits(x):
    return lax.bitcast_convert_type(
        x.astype(jnp.bfloat16), jnp.uint16).astype(jnp.int32)


def _prep_body(a_ref, bb_ref, d0_ref, d1_ref, d2_ref, t_ref,
               a_out, b0_out, b1_out, b2_out, b3_out, t2_out, prior_out):
    a = _softplus(a_ref[...])
    b0 = bb_ref[...]
    b1 = b0 + _softplus(d0_ref[...])
    b2 = b1 + _softplus(d1_ref[...])
    b3 = b2 + _softplus(d2_ref[...])
    a_out[...] = a
    b0_out[...] = b0
    b1_out[...] = b1
    b2_out[...] = b2
    b3_out[...] = b3
    t = t_ref[...]
    t2_out[...] = _bf16_bits(t[:_HALF]) | (_bf16_bits(t[_HALF:]) << 16)
    n_elem = N_ITEMS + N_ITEMS * (N_GRADES - 1) + N_PERSONS
    sq = (jnp.sum(a * a) + jnp.sum(b0 * b0) + jnp.sum(b1 * b1)
          + jnp.sum(b2 * b2) + jnp.sum(b3 * b3) + jnp.sum(t * t))
    prior_out[0, 0] = -0.5 * _LOG2PI * n_elem - 0.5 * sq


def _prep(a_, b_base_, b_diff_, t):
    vec = jax.ShapeDtypeStruct((N_ITEMS,), jnp.float32)
    return pl.pallas_call(
        _prep_body,
        out_shape=(vec, vec, vec, vec, vec,
                   jax.ShapeDtypeStruct((_HALF,), jnp.int32),
                   jax.ShapeDtypeStruct((1, 1), jnp.float32)),
        out_specs=(pl.BlockSpec(), pl.BlockSpec(), pl.BlockSpec(),
                   pl.BlockSpec(), pl.BlockSpec(), pl.BlockSpec(),
                   pl.BlockSpec(memory_space=pltpu.SMEM)),
    )(a_, b_base_.reshape(N_ITEMS),
      b_diff_[:, 0], b_diff_[:, 1], b_diff_[:, 2], t)


def _make_sc_body(n_resp, with_dep):
    per_w = n_resp // NW
    n_ch = per_w // CH

    def _sc_body(*refs):
        if with_dep:
            refs = refs[1:]
        (ir_h, pe_h, a_h, b0_h, b1_h, b2_h, b3_h, t2_h, p_h,
         a_v, b_v, t2_v, ir0, ir1, pe0, pe1, pb0, pb1,
         si0, si1, sp0, sp1) = refs
        wid = lax.axis_index("s") * NC + lax.axis_index("c")
        pltpu.sync_copy(a_h, a_v)
        for k, bk in enumerate((b0_h, b1_h, b2_h, b3_h)):
            pltpu.sync_copy(bk, b_v.at[pl.ds(k * N_ITEMS, N_ITEMS)])
        pltpu.sync_copy(t2_h, t2_v)
        base = wid * per_w

        ibufs = ((ir0, pe0), (ir1, pe1))
        pbufs = (pb0, pb1)
        isems = (si0, si1)
        psems = (sp0, sp1)

        def fire_idx(ci, b):
            off = base + lax.rem(ci, n_ch) * CH
            for src, dst in zip((ir_h, pe_h), ibufs[b]):
                pltpu.async_copy(src.at[pl.ds(off, CH)], dst, isems[b])

        fire_idx(jnp.int32(0), 0)
        fire_idx(jnp.int32(1), 1)

        def pair(k, carry):
            for b in range(2):
                ci = 2 * k + b
                (ir_v, pe_v), pb = ibufs[b], pbufs[b]
                for src, dst in zip((ir_h, pe_h), ibufs[b]):
                    pltpu.make_async_copy(
                        src.at[pl.ds(0, CH)], dst, isems[b]).wait()
                @pl.when(k >= 1)
                def _():
                    pltpu.make_async_copy(
                        pb, p_h.at[pl.ds(0, CH)], psems[b]).wait()

                def prob16(sl):
                    ir = ir_v[sl]
                    pe = pe_v[sl]
                    it = ir >> 3
                    rs = ir & 7
                    hi = pe >= _HALF
                    tw = plsc.load_gather(
                        t2_v, [pe - jnp.where(hi, _HALF, 0)])
                    bits = jnp.where(hi, tw & jnp.int32(-65536), tw << 16)
                    tv = plsc.bitcast(bits, jnp.float32)
                    av = plsc.load_gather(a_v, [it])
                    bi = it + rs * N_ITEMS
                    bu = plsc.load_gather(
                        b_v, [jnp.maximum(bi - 2 * N_ITEMS, 0)])
                    bl = plsc.load_gather(
                        b_v, [jnp.minimum(bi - N_ITEMS, N_ITEMS * 4 - 1)])
                    su = 1.0 / (1.0 + jnp.exp(av * (bu - tv)))
                    slo = 1.0 / (1.0 + jnp.exp(av * (bl - tv)))
                    upper = jnp.where(rs == 1, 1.0, su)
                    lower = jnp.where(rs == N_GRADES, 0.0, slo)
                    return jnp.clip(upper - lower, 1e-12, 1.0)

                @plsc.parallel_loop(0, CH // (2 * L), unroll=2)
                def step(s):
                    p0 = prob16(pl.ds(s * (2 * L), L))
                    p1 = prob16(pl.ds(s * (2 * L) + L, L))
                    pb[pl.ds(s * (2 * L), 2 * L)] = plsc.pack(
                        p0, p1, format=plsc.PackFormat.INTERLEAVED)

                pltpu.async_copy(
                    pb, p_h.at[pl.ds(base + ci * CH, CH)], psems[b])
                fire_idx(ci + 2, b)
            return carry

        lax.fori_loop(0, n_ch // 2, pair, 0)

        for b in range(2):
            pltpu.make_async_copy(
                pbufs[b], p_h.at[pl.ds(0, CH)], psems[b]).wait()
            for src, dst in zip((ir_h, pe_h), ibufs[b]):
                pltpu.make_async_copy(
                    src.at[pl.ds(0, CH)], dst, isems[b]).wait()

    return _sc_body


@functools.cache
def _make_sc_gather(n_resp, with_dep=False):
    mesh = plsc.VectorSubcoreMesh(
        core_axis_name="c", subcore_axis_name="s",
        num_cores=NC, num_subcores=NS)
    return functools.partial(
        pl.kernel,
        out_type=jax.ShapeDtypeStruct((n_resp,), jnp.bfloat16),
        mesh=mesh,
        scratch_types=[
            pltpu.VMEM((N_ITEMS,), jnp.float32),
            pltpu.VMEM((N_ITEMS * 4,), jnp.float32),
            pltpu.VMEM((N_PERSONS // 2,), jnp.int32),
            pltpu.VMEM((CH,), jnp.int32),
            pltpu.VMEM((CH,), jnp.int32),
            pltpu.VMEM((CH,), jnp.int32),
            pltpu.VMEM((CH,), jnp.int32),
            pltpu.VMEM((CH,), jnp.bfloat16),
            pltpu.VMEM((CH,), jnp.bfloat16),
            pltpu.SemaphoreType.DMA,
            pltpu.SemaphoreType.DMA,
            pltpu.SemaphoreType.DMA,
            pltpu.SemaphoreType.DMA,
        ],
        compiler_params=pltpu.CompilerParams(needs_layout_passes=False),
    )(_make_sc_body(n_resp, with_dep))


_RG = 16


def _reduce_body(p_ref, prior_ref, out_ref, acc_ref):
    i = pl.program_id(0)

    @pl.when(i == 0)
    def _():
        acc_ref[0] = 0.0

    acc_ref[0] += jnp.sum(jnp.log(p_ref[...].astype(jnp.float32)))

    @pl.when(i == _RG - 1)
    def _():
        out_ref[0, 0] = acc_ref[0] + prior_ref[0, 0]


def _reduce(p2d, prior):
    rows = p2d.shape[0] // _RG
    return pl.pallas_call(
        _reduce_body,
        grid=(_RG,),
        out_shape=jax.ShapeDtypeStruct((1, 1), jnp.float32),
        in_specs=(
            pl.BlockSpec((rows, 128), lambda i: (i, 0)),
            pl.BlockSpec(memory_space=pltpu.SMEM),
        ),
        out_specs=pl.BlockSpec(memory_space=pltpu.SMEM),
        scratch_shapes=[pltpu.SMEM((1,), jnp.float32)],
    )(p2d, prior)


def kernel(indices, a_, b_base_, b_diff_, t):
    itemresp = (indices[:, 0] << 3) | indices[:, 2]
    person = indices[:, 1]
    a, b0, b1, b2, b3, t2, prior = _prep(a_, b_base_, b_diff_, t)
    p = _make_sc_gather(N_RESP)(itemresp, person, a, b0, b1, b2, b3, t2)
    out = _reduce(p.reshape(N_RESP // 128, 128), prior)
    return -out.reshape(())


# speedup vs baseline: 1.2015x; 1.0010x over previous
import functools

import jax
import jax.numpy as jnp
from jax import lax
from jax.experimental import pallas as pl
from jax.experimental.pallas import tpu as pltpu
from jax.experimental.pallas import tpu_sc as plsc

N_ITEMS = 10000
N_PERSONS = 100000
N_GRADES = 5
N_RESP = 1048576

NC, NS, L = 2, 16, 16
NW = NC * NS
CH = 2048
_HALF = N_PERSONS // 2

_LOG2PI = 1.8378770664093453


def _softplus(x):
    return jnp.maximum(x, 0.0) + jnp.log1p(jnp.exp(-jnp.abs(x)))


def _bf16_bits(x):
    return lax.bitcast_convert_type(
        x.astype(jnp.bfloat16), jnp.uint16).astype(jnp.int32)


def _prep_body(a_ref, bb_ref, d0_ref, d1_ref, d2_ref, t_ref,
               a_out, b0_out, b1_out, b2_out, b3_out, t2_out, prior_out):
    a = _softplus(a_ref[...])
    b0 = bb_ref[...]
    b1 = b0 + _softplus(d0_ref[...])
    b2 = b1 + _softplus(d1_ref[...])
    b3 = b2 + _softplus(d2_ref[...])
    a_out[...] = a
    b0_out[...] = b0
    b1_out[...] = b1
    b2_out[...] = b2
    b3_out[...] = b3
    t = t_ref[...]
    t2_out[...] = _bf16_bits(t[:_HALF]) | (_bf16_bits(t[_HALF:]) << 16)
    n_elem = N_ITEMS + N_ITEMS * (N_GRADES - 1) + N_PERSONS
    sq = (jnp.sum(a * a) + jnp.sum(b0 * b0) + jnp.sum(b1 * b1)
          + jnp.sum(b2 * b2) + jnp.sum(b3 * b3) + jnp.sum(t * t))
    prior_out[0, 0] = -0.5 * _LOG2PI * n_elem - 0.5 * sq


def _prep(a_, b_base_, b_diff_, t):
    vec = jax.ShapeDtypeStruct((N_ITEMS,), jnp.float32)
    return pl.pallas_call(
        _prep_body,
        out_shape=(vec, vec, vec, vec, vec,
                   jax.ShapeDtypeStruct((_HALF,), jnp.int32),
                   jax.ShapeDtypeStruct((1, 1), jnp.float32)),
        out_specs=(pl.BlockSpec(), pl.BlockSpec(), pl.BlockSpec(),
                   pl.BlockSpec(), pl.BlockSpec(), pl.BlockSpec(),
                   pl.BlockSpec(memory_space=pltpu.SMEM)),
    )(a_, b_base_.reshape(N_ITEMS),
      b_diff_[:, 0], b_diff_[:, 1], b_diff_[:, 2], t)


def _make_sc_body(n_resp):
    per_w = n_resp // NW
    n_ch = per_w // CH

    def _sc_body(ir_h, pe_h, a_h, b0_h, b1_h, b2_h, b3_h, t2_h, p_h,
                 a_v, b_v, t2_v, ir0, ir1, pe0, pe1, pb0, pb1,
                 si0, si1, sp0, sp1):
        wid = lax.axis_index("s") * NC + lax.axis_index("c")
        pltpu.sync_copy(a_h, a_v)
        for k, bk in enumerate((b0_h, b1_h, b2_h, b3_h)):
            pltpu.sync_copy(bk, b_v.at[pl.ds(k * N_ITEMS, N_ITEMS)])
        pltpu.sync_copy(t2_h, t2_v)
        base = wid * per_w

        ibufs = ((ir0, pe0), (ir1, pe1))
        pbufs = (pb0, pb1)
        isems = (si0, si1)
        psems = (sp0, sp1)

        def fire_idx(ci, b):
            off = base + lax.rem(ci, n_ch) * CH
            for src, dst in zip((ir_h, pe_h), ibufs[b]):
                pltpu.async_copy(src.at[pl.ds(off, CH)], dst, isems[b])

        fire_idx(jnp.int32(0), 0)
        fire_idx(jnp.int32(1), 1)

        def pair(k, carry):
            for b in range(2):
                ci = 2 * k + b
                (ir_v, pe_v), pb = ibufs[b], pbufs[b]
                for src, dst in zip((ir_h, pe_h), ibufs[b]):
                    pltpu.make_async_copy(
                        src.at[pl.ds(0, CH)], dst, isems[b]).wait()
                @pl.when(k >= 1)
                def _():
                    pltpu.make_async_copy(
                        pb, p_h.at[pl.ds(0, CH)], psems[b]).wait()

                def prob16(sl):
                    ir = ir_v[sl]
                    pe = pe_v[sl]
                    it = ir >> 3
                    rs = ir & 7
                    hi = pe >= _HALF
                    tw = plsc.load_gather(
                        t2_v, [pe - jnp.where(hi, _HALF, 0)])
                    bits = jnp.where(hi, tw & jnp.int32(-65536), tw << 16)
                    tv = plsc.bitcast(bits, jnp.float32)
                    av = plsc.load_gather(a_v, [it])
                    bi = it + rs * N_ITEMS
                    bu = plsc.load_gather(
                        b_v, [jnp.maximum(bi - 2 * N_ITEMS, 0)])
                    bl = plsc.load_gather(
                        b_v, [jnp.minimum(bi - N_ITEMS, N_ITEMS * 4 - 1)])
                    su = 1.0 / (1.0 + jnp.exp(av * (bu - tv)))
                    slo = 1.0 / (1.0 + jnp.exp(av * (bl - tv)))
                    upper = jnp.where(rs == 1, 1.0, su)
                    lower = jnp.where(rs == N_GRADES, 0.0, slo)
                    return jnp.clip(upper - lower, 1e-12, 1.0)

                @plsc.parallel_loop(0, CH // (2 * L), unroll=2)
                def step(s):
                    p0 = prob16(pl.ds(s * (2 * L), L))
                    p1 = prob16(pl.ds(s * (2 * L) + L, L))
                    pb[pl.ds(s * (2 * L), 2 * L)] = plsc.pack(
                        p0, p1, format=plsc.PackFormat.INTERLEAVED)

                pltpu.async_copy(
                    pb, p_h.at[pl.ds(base + ci * CH, CH)], psems[b])
                fire_idx(ci + 2, b)
            return carry

        lax.fori_loop(0, n_ch // 2, pair, 0)

        for b in range(2):
            pltpu.make_async_copy(
                pbufs[b], p_h.at[pl.ds(0, CH)], psems[b]).wait()
            for src, dst in zip((ir_h, pe_h), ibufs[b]):
                pltpu.make_async_copy(
                    src.at[pl.ds(0, CH)], dst, isems[b]).wait()

    return _sc_body


@functools.cache
def _make_sc_gather(n_resp):
    mesh = plsc.VectorSubcoreMesh(
        core_axis_name="c", subcore_axis_name="s",
        num_cores=NC, num_subcores=NS)
    return functools.partial(
        pl.kernel,
        out_type=jax.ShapeDtypeStruct((n_resp,), jnp.bfloat16),
        mesh=mesh,
        scratch_types=[
            pltpu.VMEM((N_ITEMS,), jnp.float32),
            pltpu.VMEM((N_ITEMS * 4,), jnp.float32),
            pltpu.VMEM((N_PERSONS // 2,), jnp.int32),
            pltpu.VMEM((CH,), jnp.int32),
            pltpu.VMEM((CH,), jnp.int32),
            pltpu.VMEM((CH,), jnp.int32),
            pltpu.VMEM((CH,), jnp.int32),
            pltpu.VMEM((CH,), jnp.bfloat16),
            pltpu.VMEM((CH,), jnp.bfloat16),
            pltpu.SemaphoreType.DMA,
            pltpu.SemaphoreType.DMA,
            pltpu.SemaphoreType.DMA,
            pltpu.SemaphoreType.DMA,
        ],
        compiler_params=pltpu.CompilerParams(needs_layout_passes=False),
    )(_make_sc_body(n_resp))


_RG = 16


def _reduce_body(p_ref, prior_ref, out_ref, acc_ref):
    i = pl.program_id(0)

    @pl.when(i == 0)
    def _():
        acc_ref[0] = 0.0

    acc_ref[0] += jnp.sum(jnp.log(p_ref[...].astype(jnp.float32)))

    @pl.when(i == _RG - 1)
    def _():
        out_ref[0, 0] = acc_ref[0] + prior_ref[0, 0]


def _reduce(p2d, prior):
    rows = p2d.shape[0] // _RG
    return pl.pallas_call(
        _reduce_body,
        grid=(_RG,),
        out_shape=jax.ShapeDtypeStruct((1, 1), jnp.float32),
        in_specs=(
            pl.BlockSpec((rows, 128), lambda i: (i, 0)),
            pl.BlockSpec(memory_space=pltpu.SMEM),
        ),
        out_specs=pl.BlockSpec(memory_space=pltpu.SMEM),
        scratch_shapes=[pltpu.SMEM((1,), jnp.float32)],
    )(p2d, prior)


def kernel(indices, a_, b_base_, b_diff_, t):
    itemresp = (indices[:, 0] << 3) | indices[:, 2]
    person = indices[:, 1]
    a, b0, b1, b2, b3, t2, prior = _prep(a_, b_base_, b_diff_, t)
    p = _make_sc_gather(N_RESP)(itemresp, person, a, b0, b1, b2, b3, t2)
    out = _reduce(p.reshape(N_RESP // 128, 128), prior)
    return -out.reshape(())
